# Initial kernel scaffold; baseline (speedup 1.0000x reference)
import functools

import jax
import jax.numpy as jnp
from jax import lax
from jax.experimental import pallas as pl
from jax.experimental.pallas import tpu as pltpu
from jax.experimental.pallas import tpu_sc as plsc

N = 10000
D_IN = 128
D = 4
E = 320000

NC = 2
NS = 16
NW = NC * NS
CHUNK = 128
CPT = 79
EPAD = NW * CPT * CHUNK
NPAD = 10240
RPS = NPAD // NS
NTAIL = NPAD - N

_mesh = plsc.VectorSubcoreMesh(core_axis_name="c", subcore_axis_name="s")


@functools.partial(
    pl.kernel,
    out_type=jax.ShapeDtypeStruct((NC, NPAD), jnp.float32),
    mesh=_mesh,
    scratch_types=[
        pltpu.VMEM((CPT, CHUNK), jnp.int32),
        pltpu.VMEM((CHUNK,), jnp.float32),
        pltpu.VMEM_SHARED((NPAD,), jnp.float32),
        pltpu.SemaphoreType.DMA,
    ],
)
def _deg_kernel(dst_hbm, zeros_hbm, out_hbm, dst_v, ones_v, acc_sh, sem):
    cid = lax.axis_index("c")
    sid = lax.axis_index("s")
    w = sid * NC + cid
    pltpu.sync_copy(zeros_hbm.at[pl.ds(sid * RPS, RPS)],
                    acc_sh.at[pl.ds(sid * RPS, RPS)])

    def _fill(i, carry):
        ones_v[pl.ds(i * 16, 16)] = jnp.full((16,), 1.0, jnp.float32)
        return carry
    lax.fori_loop(0, CHUNK // 16, _fill, 0)

    pltpu.sync_copy(dst_hbm.at[pl.ds(w * CPT, CPT), :], dst_v)
    plsc.subcore_barrier()

    def _fire(j, carry):
        pltpu.make_async_copy(ones_v, acc_sh.at[dst_v.at[j]], sem).start(add=True)
        return carry
    lax.fori_loop(0, CPT, _fire, 0)

    def _drain(j, carry):
        pltpu.make_async_copy(ones_v, acc_sh.at[dst_v.at[0]], sem).wait()
        return carry
    lax.fori_loop(0, CPT, _drain, 0)
    plsc.subcore_barrier()
    pltpu.sync_copy(acc_sh.at[pl.ds(sid * RPS, RPS)],
                    out_hbm.at[cid, pl.ds(sid * RPS, RPS)])


@functools.partial(
    pl.kernel,
    out_type=jax.ShapeDtypeStruct((NC, NPAD, D), jnp.float32),
    mesh=_mesh,
    scratch_types=[
        pltpu.VMEM((CPT, CHUNK), jnp.int32),
        pltpu.VMEM((CPT, CHUNK), jnp.int32),
        pltpu.VMEM((CPT, CHUNK, D), jnp.float32),
        pltpu.VMEM_SHARED((NPAD, D), jnp.float32),
        pltpu.SemaphoreType.DMA,
        pltpu.SemaphoreType.DMA,
    ],
)
def _agg_kernel(g_hbm, src_hbm, dst_hbm, zeros_hbm, out_hbm,
                src_v, dst_v, rows_v, acc_sh, gsem, ssem):
    cid = lax.axis_index("c")
    sid = lax.axis_index("s")
    w = sid * NC + cid
    pltpu.sync_copy(zeros_hbm.at[pl.ds(sid * RPS, RPS), :],
                    acc_sh.at[pl.ds(sid * RPS, RPS), :])
    pltpu.sync_copy(src_hbm.at[pl.ds(w * CPT, CPT), :], src_v)
    pltpu.sync_copy(dst_hbm.at[pl.ds(w * CPT, CPT), :], dst_v)

    def _gfire(j, carry):
        pltpu.make_async_copy(g_hbm.at[src_v.at[j]], rows_v.at[j], gsem).start()
        return carry
    lax.fori_loop(0, CPT, _gfire, 0)
    plsc.subcore_barrier()

    def _sfire(j, carry):
        pltpu.make_async_copy(g_hbm.at[src_v.at[0]], rows_v.at[0], gsem).wait()
        pltpu.make_async_copy(rows_v.at[j], acc_sh.at[dst_v.at[j]], ssem).start(add=True)
        return carry
    lax.fori_loop(0, CPT, _sfire, 0)

    def _sdrain(j, carry):
        pltpu.make_async_copy(rows_v.at[0], acc_sh.at[dst_v.at[0]], ssem).wait()
        return carry
    lax.fori_loop(0, CPT, _sdrain, 0)
    plsc.subcore_barrier()
    pltpu.sync_copy(acc_sh.at[pl.ds(sid * RPS, RPS), :],
                    out_hbm.at[cid, pl.ds(sid * RPS, RPS), :])


_BLK = 1024
_GRID = NPAD // _BLK


def _tc_a_body(dp_ref, x_ref, w_ref, g_ref, dinv_ref):
    deg = dp_ref[0] + dp_ref[1] + 1.0
    dinv = lax.rsqrt(deg)
    h = jnp.dot(x_ref[...], w_ref[...], preferred_element_type=jnp.float32)
    g_ref[...] = h * dinv
    dinv_ref[...] = dinv


def _tc_a(degp, x_pad, W1):
    return pl.pallas_call(
        _tc_a_body,
        grid=(_GRID,),
        in_specs=[
            pl.BlockSpec((2, _BLK, 1), lambda i: (0, i, 0)),
            pl.BlockSpec((_BLK, D_IN), lambda i: (i, 0)),
            pl.BlockSpec((D_IN, D), lambda i: (0, 0)),
        ],
        out_specs=[
            pl.BlockSpec((_BLK, D), lambda i: (i, 0)),
            pl.BlockSpec((_BLK, 1), lambda i: (i, 0)),
        ],
        out_shape=[
            jax.ShapeDtypeStruct((NPAD, D), jnp.float32),
            jax.ShapeDtypeStruct((NPAD, 1), jnp.float32),
        ],
    )(degp, x_pad, W1)


def _tc_b_body(p_ref, g1_ref, dinv_ref, w2_ref, b1_ref, g2_ref):
    s = p_ref[0] + p_ref[1] + g1_ref[...]
    h = jnp.tanh(s * dinv_ref[...] + b1_ref[...])
    acc = h[:, 0:1] * w2_ref[0:1, :]
    acc += h[:, 1:2] * w2_ref[1:2, :]
    acc += h[:, 2:3] * w2_ref[2:3, :]
    acc += h[:, 3:4] * w2_ref[3:4, :]
    g2_ref[...] = acc * dinv_ref[...]


def _tc_b(p, g1, dinv, W2, b1):
    return pl.pallas_call(
        _tc_b_body,
        grid=(_GRID,),
        in_specs=[
            pl.BlockSpec((2, _BLK, D), lambda i: (0, i, 0)),
            pl.BlockSpec((_BLK, D), lambda i: (i, 0)),
            pl.BlockSpec((_BLK, 1), lambda i: (i, 0)),
            pl.BlockSpec((D, D), lambda i: (0, 0)),
            pl.BlockSpec((1, D), lambda i: (0, 0)),
        ],
        out_specs=pl.BlockSpec((_BLK, D), lambda i: (i, 0)),
        out_shape=jax.ShapeDtypeStruct((NPAD, D), jnp.float32),
    )(p, g1, dinv, W2, b1)


def _tc_c_body(q_ref, g2_ref, dinv_ref, b2_ref, o_ref, ls_ref):
    o = (q_ref[0] + q_ref[1] + g2_ref[...]) * dinv_ref[...] + b2_ref[...]
    m = jnp.max(o, axis=1, keepdims=True)
    z = o - m
    lse = jnp.log(jnp.sum(jnp.exp(z), axis=1, keepdims=True))
    o_ref[...] = o
    ls_ref[...] = z - lse


def _tc_c(q, g2, dinv, b2):
    return pl.pallas_call(
        _tc_c_body,
        grid=(_GRID,),
        in_specs=[
            pl.BlockSpec((2, _BLK, D), lambda i: (0, i, 0)),
            pl.BlockSpec((_BLK, D), lambda i: (i, 0)),
            pl.BlockSpec((_BLK, 1), lambda i: (i, 0)),
            pl.BlockSpec((1, D), lambda i: (0, 0)),
        ],
        out_specs=[
            pl.BlockSpec((_BLK, D), lambda i: (i, 0)),
            pl.BlockSpec((_BLK, D), lambda i: (i, 0)),
        ],
        out_shape=[
            jax.ShapeDtypeStruct((NPAD, D), jnp.float32),
            jax.ShapeDtypeStruct((NPAD, D), jnp.float32),
        ],
    )(q, g2, dinv, b2)


def kernel(x, edge_index, W1, b1, W2, b2):
    x_pad = jnp.pad(x, ((0, NPAD - N), (0, 0)))
    padi = (jnp.arange(EPAD - E, dtype=jnp.int32) % NTAIL) + N
    src = jnp.concatenate([edge_index[0], padi]).reshape(NW * CPT, CHUNK)
    dst = jnp.concatenate([edge_index[1], padi]).reshape(NW * CPT, CHUNK)
    zeros1 = jnp.zeros((NPAD,), jnp.float32)
    zeros4 = jnp.zeros((NPAD, D), jnp.float32)

    degp = _deg_kernel(dst, zeros1)
    g1, dinv = _tc_a(degp[:, :, None], x_pad, W1)
    p = _agg_kernel(g1, src, dst, zeros4)
    g2 = _tc_b(p, g1, dinv, W2, b1[None, :])
    q = _agg_kernel(g2, src, dst, zeros4)
    o, ls = _tc_c(q, g2, dinv, b2[None, :])
    return (o[:N], ls[:N])


# speedup vs baseline: 27.5212x; 27.5212x over previous
import functools

import jax
import jax.numpy as jnp
from jax import lax
from jax.experimental import pallas as pl
from jax.experimental.pallas import tpu as pltpu
from jax.experimental.pallas import tpu_sc as plsc

N = 10000
D_IN = 128
D = 4
E = 320000

NC = 2
NS = 16
NW = NC * NS
EPT = 10240
EPAD = NW * EPT
NPAD = 10240
SRN = NPAD // NS
NTAIL = NPAD - N

_mesh = plsc.VectorSubcoreMesh(core_axis_name="c", subcore_axis_name="s")
_sc_params = pltpu.CompilerParams(use_tc_tiling_on_sc=False)


@functools.partial(
    pl.kernel,
    out_type=jax.ShapeDtypeStruct((NC, NPAD), jnp.float32),
    mesh=_mesh,
    compiler_params=_sc_params,
    scratch_types=[
        pltpu.VMEM((EPT,), jnp.int32),
        pltpu.VMEM((EPT,), jnp.float32),
        pltpu.VMEM_SHARED((NPAD,), jnp.float32),
        pltpu.SemaphoreType.DMA,
    ],
)
def _deg_kernel(dst_hbm, ones_hbm, zeros_hbm, out_hbm, dst_v, ones_v, acc_sh, sem):
    cid = lax.axis_index("c")
    sid = lax.axis_index("s")
    w = sid * NC + cid
    pltpu.sync_copy(zeros_hbm.at[pl.ds(sid * SRN, SRN)],
                    acc_sh.at[pl.ds(sid * SRN, SRN)])
    pltpu.sync_copy(ones_hbm, ones_v)
    pltpu.sync_copy(dst_hbm.at[pl.ds(w * EPT, EPT)], dst_v)
    plsc.subcore_barrier()
    pltpu.make_async_copy(ones_v, acc_sh.at[dst_v], sem).start(add=True)
    pltpu.make_async_copy(ones_v, acc_sh.at[dst_v], sem).wait()
    plsc.subcore_barrier()
    pltpu.sync_copy(acc_sh.at[pl.ds(sid * SRN, SRN)],
                    out_hbm.at[cid, pl.ds(sid * SRN, SRN)])


@functools.partial(
    pl.kernel,
    out_type=[jax.ShapeDtypeStruct((NC, NPAD), jnp.float32)] * D,
    mesh=_mesh,
    compiler_params=_sc_params,
    scratch_types=[
        pltpu.VMEM((EPT,), jnp.int32),
        pltpu.VMEM((EPT,), jnp.int32),
    ] + [pltpu.VMEM((EPT,), jnp.float32)] * D + [
        pltpu.VMEM_SHARED((NPAD,), jnp.float32)
    ] * D + [
        pltpu.SemaphoreType.DMA,
        pltpu.SemaphoreType.DMA,
    ],
)
def _agg_kernel(g0, g1, g2, g3, src_hbm, dst_hbm, zeros_hbm,
                o0, o1, o2, o3,
                src_v, dst_v, v0, v1, v2, v3, a0, a1, a2, a3, gsem, ssem):
    g = (g0, g1, g2, g3)
    vals = (v0, v1, v2, v3)
    acc = (a0, a1, a2, a3)
    outs = (o0, o1, o2, o3)
    cid = lax.axis_index("c")
    sid = lax.axis_index("s")
    w = sid * NC + cid
    for j in range(D):
        pltpu.sync_copy(zeros_hbm.at[pl.ds(sid * SRN, SRN)],
                        acc[j].at[pl.ds(sid * SRN, SRN)])
    pltpu.sync_copy(src_hbm.at[pl.ds(w * EPT, EPT)], src_v)
    pltpu.sync_copy(dst_hbm.at[pl.ds(w * EPT, EPT)], dst_v)
    for j in range(D):
        pltpu.make_async_copy(g[j].at[src_v], vals[j], gsem).start()
    plsc.subcore_barrier()
    for j in range(D):
        pltpu.make_async_copy(g[j].at[src_v], vals[j], gsem).wait()
    for j in range(D):
        pltpu.make_async_copy(vals[j], acc[j].at[dst_v], ssem).start(add=True)
    for j in range(D):
        pltpu.make_async_copy(vals[j], acc[j].at[dst_v], ssem).wait()
    plsc.subcore_barrier()
    for j in range(D):
        pltpu.sync_copy(acc[j].at[pl.ds(sid * SRN, SRN)],
                        outs[j].at[cid, pl.ds(sid * SRN, SRN)])


def _tc_a_body(dp_ref, x_ref, w_ref, g_ref, dinv_ref):
    d = dp_ref[0] + dp_ref[1] + 1.0
    dinv = lax.rsqrt(d)
    h = jnp.dot(x_ref[...], w_ref[...], preferred_element_type=jnp.float32)
    g_ref[...] = h * dinv[:, None]
    dinv_ref[...] = dinv


_tc_a = pl.pallas_call(
    _tc_a_body,
    out_shape=[
        jax.ShapeDtypeStruct((NPAD, D), jnp.float32),
        jax.ShapeDtypeStruct((NPAD,), jnp.float32),
    ],
)

_PLANE2D = (NPAD // 128, 128)


def _tc_b_body(p0, p1, p2, p3, g0, g1, g2, g3, dinv_ref, w2_ref, b1_ref,
               q0, q1, q2, q3):
    dinv = dinv_ref[...]
    h = []
    for j, (p, g) in enumerate(zip((p0, p1, p2, p3), (g0, g1, g2, g3))):
        s = p[0] + p[1] + g[...]
        h.append(jnp.tanh(s * dinv + b1_ref[0:1, j:j + 1]))
    for j, q in enumerate((q0, q1, q2, q3)):
        acc = h[0] * w2_ref[0:1, j:j + 1]
        acc += h[1] * w2_ref[1:2, j:j + 1]
        acc += h[2] * w2_ref[2:3, j:j + 1]
        acc += h[3] * w2_ref[3:4, j:j + 1]
        q[...] = acc * dinv


_tc_b = pl.pallas_call(
    _tc_b_body,
    out_shape=[jax.ShapeDtypeStruct(_PLANE2D, jnp.float32)] * D,
)


def _tc_c_body(q0, q1, q2, q3, g0, g1, g2, g3, dinv_ref, b2_ref,
               o0, o1, o2, o3, l0, l1, l2, l3):
    dinv = dinv_ref[...]
    o = []
    for j, (q, g) in enumerate(zip((q0, q1, q2, q3), (g0, g1, g2, g3))):
        o.append((q[0] + q[1] + g[...]) * dinv + b2_ref[0:1, j:j + 1])
    m = jnp.maximum(jnp.maximum(o[0], o[1]), jnp.maximum(o[2], o[3]))
    z = [oj - m for oj in o]
    lse = jnp.log(jnp.exp(z[0]) + jnp.exp(z[1]) + jnp.exp(z[2]) + jnp.exp(z[3]))
    for j, (oref, lref) in enumerate(zip((o0, o1, o2, o3), (l0, l1, l2, l3))):
        oref[...] = o[j]
        lref[...] = z[j] - lse


_tc_c = pl.pallas_call(
    _tc_c_body,
    out_shape=[jax.ShapeDtypeStruct(_PLANE2D, jnp.float32)] * (2 * D),
)


def kernel(x, edge_index, W1, b1, W2, b2):
    x_pad = jnp.pad(x, ((0, NPAD - N), (0, 0)))
    padi = (jnp.arange(EPAD - E, dtype=jnp.int32) % NTAIL) + N
    src_e = jnp.concatenate([edge_index[0], padi])
    dst_e = jnp.concatenate([edge_index[1], padi])
    ones = jnp.ones((EPT,), jnp.float32)
    zeros1 = jnp.zeros((NPAD,), jnp.float32)

    degp = _deg_kernel(dst_e, ones, zeros1)
    g1nm, dinv = _tc_a(degp, x_pad, W1)
    g1t = g1nm.T
    g1p = [g1t[j] for j in range(D)]
    p = _agg_kernel(*g1p, src_e, dst_e, zeros1)
    dinv2 = dinv.reshape(_PLANE2D)
    g2p2 = _tc_b(*[a.reshape((NC,) + _PLANE2D) for a in p],
                 *[a.reshape(_PLANE2D) for a in g1p],
                 dinv2, W2, b1[None, :])
    g2p = [a.reshape(NPAD) for a in g2p2]
    q = _agg_kernel(*g2p, src_e, dst_e, zeros1)
    outs = _tc_c(*[a.reshape((NC,) + _PLANE2D) for a in q],
                 *g2p2, dinv2, b2[None, :])
    o = jnp.stack([a.reshape(NPAD) for a in outs[:D]], axis=1)[:N]
    ls = jnp.stack([a.reshape(NPAD) for a in outs[D:]], axis=1)[:N]
    return (o, ls)


# speedup vs baseline: 75.1941x; 2.7322x over previous
import functools

import jax
import jax.numpy as jnp
from jax import lax
from jax.experimental import pallas as pl
from jax.experimental.pallas import tpu as pltpu
from jax.experimental.pallas import tpu_sc as plsc

N = 10000
D_IN = 128
D = 4
E = 320000

NC = 2
NS = 16
NW = NC * NS
EPT = 10240
EPAD = NW * EPT
NPAD = 10240
SRN = NPAD // NS
NTAIL = NPAD - N

_mesh = plsc.VectorSubcoreMesh(core_axis_name="c", subcore_axis_name="s")
_sc_params = pltpu.CompilerParams(use_tc_tiling_on_sc=False,
                                  needs_layout_passes=False)


@functools.partial(
    pl.kernel,
    out_type=jax.ShapeDtypeStruct((NC, NPAD), jnp.float32),
    mesh=_mesh,
    compiler_params=_sc_params,
    scratch_types=[
        pltpu.VMEM((EPT,), jnp.int32),
        pltpu.VMEM((EPT,), jnp.float32),
        pltpu.VMEM_SHARED((NPAD,), jnp.float32),
        pltpu.SemaphoreType.DMA,
    ],
)
def _deg_kernel(dst_hbm, ones_hbm, zeros_hbm, out_hbm, dst_v, ones_v, acc_sh, sem):
    cid = lax.axis_index("c")
    sid = lax.axis_index("s")
    w = sid * NC + cid
    pltpu.sync_copy(zeros_hbm.at[pl.ds(sid * SRN, SRN)],
                    acc_sh.at[pl.ds(sid * SRN, SRN)])
    pltpu.sync_copy(ones_hbm, ones_v)
    pltpu.sync_copy(dst_hbm.at[pl.ds(w * EPT, EPT)], dst_v)
    plsc.subcore_barrier()
    pltpu.make_async_copy(ones_v, acc_sh.at[dst_v], sem).start(add=True)
    pltpu.make_async_copy(ones_v, acc_sh.at[dst_v], sem).wait()
    plsc.subcore_barrier()
    pltpu.sync_copy(acc_sh.at[pl.ds(sid * SRN, SRN)],
                    out_hbm.at[cid, pl.ds(sid * SRN, SRN)])


QC = EPT // 4
DP = 8


@functools.partial(
    pl.kernel,
    out_type=[jax.ShapeDtypeStruct((NC, NPAD), jnp.float32)] * D,
    mesh=_mesh,
    compiler_params=_sc_params,
    scratch_types=[
        pltpu.VMEM((EPT,), jnp.int32),
    ] + [pltpu.VMEM((QC,), jnp.int32)] * 4
      + [pltpu.VMEM((NPAD,), jnp.float32)] * D + [
        pltpu.VMEM((QC, DP), jnp.float32),
        pltpu.VMEM((QC, DP), jnp.float32),
        pltpu.VMEM((SRN, DP), jnp.float32),
        pltpu.VMEM((SRN,), jnp.float32),
        pltpu.VMEM_SHARED((NPAD, DP), jnp.float32),
        pltpu.SemaphoreType.DMA,
        pltpu.SemaphoreType.DMA,
        pltpu.SemaphoreType.DMA,
    ],
)
def _agg_kernel(g0, g1, g2, g3, src_hbm, dst_hbm, zeros_hbm,
                o0, o1, o2, o3,
                src_v, d0, d1, d2, d3, t0, t1, t2, t3, ra, rb, tmp_v, pb_v,
                acc_sh, gsem, sa, sb):
    gp = (g0, g1, g2, g3)
    tabs = (t0, t1, t2, t3)
    dq = (d0, d1, d2, d3)
    outs = (o0, o1, o2, o3)
    bufs = (ra, rb, ra, rb)
    sems = (sa, sb, sa, sb)
    cid = lax.axis_index("c")
    sid = lax.axis_index("s")
    w = sid * NC + cid
    pltpu.sync_copy(zeros_hbm.at[pl.ds(sid * SRN, SRN), :],
                    acc_sh.at[pl.ds(sid * SRN, SRN), :])
    for j in range(D):
        pltpu.make_async_copy(gp[j], tabs[j], gsem).start()
    pltpu.sync_copy(src_hbm.at[pl.ds(w * EPT, EPT)], src_v)
    for q in range(4):
        pltpu.sync_copy(dst_hbm.at[pl.ds(w * EPT + q * QC, QC)], dq[q])
    for j in range(D):
        pltpu.make_async_copy(gp[j], tabs[j], gsem).wait()

    iota = lax.iota(jnp.int32, 16)
    cols = [jnp.full((16,), j, jnp.int32) for j in range(D)]

    def _build(q, buf):
        def _step(i, carry):
            sv = src_v[pl.ds(q * QC + i * 16, 16)]
            rowi = iota + i * 16
            for j in range(D):
                val = plsc.load_gather(tabs[j], [sv])
                plsc.store_scatter(buf, [rowi, cols[j]], val)
            return carry
        lax.fori_loop(0, QC // 16, _step, 0)

    _build(0, ra)
    plsc.subcore_barrier()
    pltpu.make_async_copy(ra, acc_sh.at[d0], sa).start(add=True)
    _build(1, rb)
    pltpu.make_async_copy(rb, acc_sh.at[d1], sb).start(add=True)
    pltpu.make_async_copy(ra, acc_sh.at[d0], sa).wait()
    _build(2, ra)
    pltpu.make_async_copy(ra, acc_sh.at[d2], sa).start(add=True)
    pltpu.make_async_copy(rb, acc_sh.at[d1], sb).wait()
    _build(3, rb)
    pltpu.make_async_copy(rb, acc_sh.at[d3], sb).start(add=True)
    pltpu.make_async_copy(ra, acc_sh.at[d2], sa).wait()
    pltpu.make_async_copy(rb, acc_sh.at[d3], sb).wait()
    plsc.subcore_barrier()

    pltpu.sync_copy(acc_sh.at[pl.ds(sid * SRN, SRN), :], tmp_v)
    for j in range(D):
        def _unpack(k, carry, _j=j):
            rows = iota + k * 16
            val = plsc.load_gather(tmp_v, [rows, cols[_j]])
            pb_v[pl.ds(k * 16, 16)] = val
            return carry
        lax.fori_loop(0, SRN // 16, _unpack, 0)
        pltpu.sync_copy(pb_v, outs[j].at[cid, pl.ds(sid * SRN, SRN)])


def _tc_a_body(dp_ref, x_ref, w_ref, g_ref, dinv_ref):
    d = dp_ref[0] + dp_ref[1] + 1.0
    dinv = lax.rsqrt(d)
    h = jnp.dot(x_ref[...], w_ref[...], preferred_element_type=jnp.float32)
    g_ref[...] = h * dinv[:, None]
    dinv_ref[...] = dinv


_tc_a = pl.pallas_call(
    _tc_a_body,
    out_shape=[
        jax.ShapeDtypeStruct((NPAD, D), jnp.float32),
        jax.ShapeDtypeStruct((NPAD,), jnp.float32),
    ],
)

_PLANE2D = (NPAD // 128, 128)


def _tc_b_body(p0, p1, p2, p3, g0, g1, g2, g3, dinv_ref, w2_ref, b1_ref,
               q0, q1, q2, q3):
    dinv = dinv_ref[...]
    h = []
    for j, (p, g) in enumerate(zip((p0, p1, p2, p3), (g0, g1, g2, g3))):
        s = p[0] + p[1] + g[...]
        h.append(jnp.tanh(s * dinv + b1_ref[0:1, j:j + 1]))
    for j, q in enumerate((q0, q1, q2, q3)):
        acc = h[0] * w2_ref[0:1, j:j + 1]
        acc += h[1] * w2_ref[1:2, j:j + 1]
        acc += h[2] * w2_ref[2:3, j:j + 1]
        acc += h[3] * w2_ref[3:4, j:j + 1]
        q[...] = acc * dinv


_tc_b = pl.pallas_call(
    _tc_b_body,
    out_shape=[jax.ShapeDtypeStruct(_PLANE2D, jnp.float32)] * D,
)


def _tc_c_body(q0, q1, q2, q3, g0, g1, g2, g3, dinv_ref, b2_ref,
               o0, o1, o2, o3, l0, l1, l2, l3):
    dinv = dinv_ref[...]
    o = []
    for j, (q, g) in enumerate(zip((q0, q1, q2, q3), (g0, g1, g2, g3))):
        o.append((q[0] + q[1] + g[...]) * dinv + b2_ref[0:1, j:j + 1])
    m = jnp.maximum(jnp.maximum(o[0], o[1]), jnp.maximum(o[2], o[3]))
    z = [oj - m for oj in o]
    lse = jnp.log(jnp.exp(z[0]) + jnp.exp(z[1]) + jnp.exp(z[2]) + jnp.exp(z[3]))
    for j, (oref, lref) in enumerate(zip((o0, o1, o2, o3), (l0, l1, l2, l3))):
        oref[...] = o[j]
        lref[...] = z[j] - lse


_tc_c = pl.pallas_call(
    _tc_c_body,
    out_shape=[jax.ShapeDtypeStruct(_PLANE2D, jnp.float32)] * (2 * D),
)


def kernel(x, edge_index, W1, b1, W2, b2):
    x_pad = jnp.pad(x, ((0, NPAD - N), (0, 0)))
    padi = (jnp.arange(EPAD - E, dtype=jnp.int32) % NTAIL) + N
    src_e = jnp.concatenate([edge_index[0], padi])
    dst_e = jnp.concatenate([edge_index[1], padi])
    ones = jnp.ones((EPT,), jnp.float32)
    zeros1 = jnp.zeros((NPAD,), jnp.float32)
    zeros8 = jnp.zeros((NPAD, DP), jnp.float32)

    degp = _deg_kernel(dst_e, ones, zeros1)
    g1nm, dinv = _tc_a(degp, x_pad, W1)
    g1t = g1nm.T
    g1p = [g1t[j] for j in range(D)]
    p = _agg_kernel(*g1p, src_e, dst_e, zeros8)
    dinv2 = dinv.reshape(_PLANE2D)
    g2p2 = _tc_b(*[a.reshape((NC,) + _PLANE2D) for a in p],
                 *[a.reshape(_PLANE2D) for a in g1p],
                 dinv2, W2, b1[None, :])
    g2p = [a.reshape(NPAD) for a in g2p2]
    q = _agg_kernel(*g2p, src_e, dst_e, zeros8)
    outs = _tc_c(*[a.reshape((NC,) + _PLANE2D) for a in q],
                 *g2p2, dinv2, b2[None, :])
    o = jnp.stack([a.reshape(NPAD) for a in outs[:D]], axis=1)[:N]
    ls = jnp.stack([a.reshape(NPAD) for a in outs[D:]], axis=1)[:N]
    return (o, ls)


# speedup vs baseline: 86.1984x; 1.1463x over previous
import functools

import jax
import jax.numpy as jnp
from jax import lax
from jax.experimental import pallas as pl
from jax.experimental.pallas import tpu as pltpu
from jax.experimental.pallas import tpu_sc as plsc

N = 10000
D_IN = 128
D = 4
E = 320000

NC = 2
NS = 16
NW = NC * NS
EPT = 10240
EPAD = NW * EPT
NPAD = 10240
PR = NPAD // 128
SRN = NPAD // NS
NTAIL = NPAD - N
QC = EPT // 4
DP = 8

_mesh = plsc.VectorSubcoreMesh(core_axis_name="c", subcore_axis_name="s")
_sc_params = pltpu.CompilerParams(use_tc_tiling_on_sc=False,
                                  needs_layout_passes=False)


@functools.partial(
    pl.kernel,
    out_type=jax.ShapeDtypeStruct((NC, NPAD), jnp.float32),
    mesh=_mesh,
    compiler_params=_sc_params,
    scratch_types=[
        pltpu.VMEM((EPT,), jnp.int32),
        pltpu.VMEM((EPT,), jnp.float32),
        pltpu.VMEM_SHARED((NPAD,), jnp.float32),
        pltpu.SemaphoreType.DMA,
    ],
)
def _deg_kernel(dst_hbm, ones_hbm, zeros_hbm, out_hbm, dst_v, ones_v, acc_sh, sem):
    cid = lax.axis_index("c")
    sid = lax.axis_index("s")
    w = sid * NC + cid
    pltpu.sync_copy(zeros_hbm.at[pl.ds(sid * SRN, SRN)],
                    acc_sh.at[pl.ds(sid * SRN, SRN)])
    pltpu.sync_copy(ones_hbm, ones_v)
    pltpu.sync_copy(dst_hbm.at[pl.ds(w * EPT, EPT)], dst_v)
    plsc.subcore_barrier()
    pltpu.make_async_copy(ones_v, acc_sh.at[dst_v], sem).start(add=True)
    pltpu.make_async_copy(ones_v, acc_sh.at[dst_v], sem).wait()
    plsc.subcore_barrier()
    pltpu.sync_copy(acc_sh.at[pl.ds(sid * SRN, SRN)],
                    out_hbm.at[cid, pl.ds(sid * SRN, SRN)])


@functools.partial(
    pl.kernel,
    out_type=[jax.ShapeDtypeStruct((NC, PR, 128), jnp.float32)] * D,
    mesh=_mesh,
    compiler_params=_sc_params,
    scratch_types=[
        pltpu.VMEM((EPT,), jnp.int32),
    ] + [pltpu.VMEM((QC,), jnp.int32)] * 4
      + [pltpu.VMEM((PR, 128), jnp.float32)] * D + [
        pltpu.VMEM((QC, DP), jnp.float32),
        pltpu.VMEM((QC, DP), jnp.float32),
        pltpu.VMEM((SRN, DP), jnp.float32),
        pltpu.VMEM((SRN // 128, 128), jnp.float32),
        pltpu.VMEM_SHARED((NPAD, DP), jnp.float32),
        pltpu.SemaphoreType.DMA,
        pltpu.SemaphoreType.DMA,
        pltpu.SemaphoreType.DMA,
    ],
)
def _agg_kernel(g_hbm, src_hbm, dst_hbm, zeros_hbm,
                o0, o1, o2, o3,
                src_v, d0, d1, d2, d3, t0, t1, t2, t3, ra, rb, tmp_v, pb_v,
                acc_sh, gsem, sa, sb):
    tabs = (t0, t1, t2, t3)
    outs = (o0, o1, o2, o3)
    cid = lax.axis_index("c")
    sid = lax.axis_index("s")
    w = sid * NC + cid
    pltpu.sync_copy(zeros_hbm.at[pl.ds(sid * SRN, SRN), :],
                    acc_sh.at[pl.ds(sid * SRN, SRN), :])
    for j in range(D):
        pltpu.make_async_copy(g_hbm.at[j], tabs[j], gsem).start()
    pltpu.sync_copy(src_hbm.at[pl.ds(w * EPT, EPT)], src_v)
    for q in range(4):
        pltpu.sync_copy(dst_hbm.at[pl.ds(w * EPT + q * QC, QC)], (d0, d1, d2, d3)[q])
    for j in range(D):
        pltpu.make_async_copy(g_hbm.at[j], tabs[j], gsem).wait()

    iota = lax.iota(jnp.int32, 16)
    cols = [jnp.full((16,), j, jnp.int32) for j in range(D)]

    def _build(q, buf):
        def _step(i, carry):
            sv = src_v[pl.ds(q * QC + i * 16, 16)]
            srow = lax.shift_right_logical(sv, 7)
            scol = lax.bitwise_and(sv, 127)
            rowi = iota + i * 16
            for j in range(D):
                val = plsc.load_gather(tabs[j], [srow, scol])
                plsc.store_scatter(buf, [rowi, cols[j]], val)
            return carry
        lax.fori_loop(0, QC // 16, _step, 0)

    _build(0, ra)
    plsc.subcore_barrier()
    pltpu.make_async_copy(ra, acc_sh.at[d0], sa).start(add=True)
    _build(1, rb)
    pltpu.make_async_copy(rb, acc_sh.at[d1], sb).start(add=True)
    pltpu.make_async_copy(ra, acc_sh.at[d0], sa).wait()
    _build(2, ra)
    pltpu.make_async_copy(ra, acc_sh.at[d2], sa).start(add=True)
    pltpu.make_async_copy(rb, acc_sh.at[d1], sb).wait()
    _build(3, rb)
    pltpu.make_async_copy(rb, acc_sh.at[d3], sb).start(add=True)
    pltpu.make_async_copy(ra, acc_sh.at[d2], sa).wait()
    pltpu.make_async_copy(rb, acc_sh.at[d3], sb).wait()
    plsc.subcore_barrier()

    pltpu.sync_copy(acc_sh.at[pl.ds(sid * SRN, SRN), :], tmp_v)
    for j in range(D):
        for k in range(SRN // 16):
            rows = iota + k * 16
            val = plsc.load_gather(tmp_v, [rows, cols[j]])
            pb_v[k // 8, pl.ds((k % 8) * 16, 16)] = val
        pltpu.sync_copy(pb_v, outs[j].at[cid, pl.ds(sid * (SRN // 128), SRN // 128), :])


def _tc_a_body(dp_ref, x_ref, w1t_ref, g_ref, dinv_ref):
    dinv = lax.rsqrt(dp_ref[0] + dp_ref[1] + 1.0)
    ht = lax.dot_general(w1t_ref[...], x_ref[...],
                         (((1,), (2,)), ((), ())),
                         preferred_element_type=jnp.float32)
    g_ref[...] = ht * dinv
    dinv_ref[...] = dinv


_tc_a = pl.pallas_call(
    _tc_a_body,
    out_shape=[
        jax.ShapeDtypeStruct((D, PR, 128), jnp.float32),
        jax.ShapeDtypeStruct((PR, 128), jnp.float32),
    ],
)


def _tc_b_body(p0, p1, p2, p3, g_ref, dinv_ref, w2_ref, b1_ref, q_ref):
    dinv = dinv_ref[...]
    h = []
    for j, p in enumerate((p0, p1, p2, p3)):
        s = p[0] + p[1] + g_ref[j]
        h.append(jnp.tanh(s * dinv + b1_ref[0:1, j:j + 1]))
    for j in range(D):
        acc = h[0] * w2_ref[0:1, j:j + 1]
        acc += h[1] * w2_ref[1:2, j:j + 1]
        acc += h[2] * w2_ref[2:3, j:j + 1]
        acc += h[3] * w2_ref[3:4, j:j + 1]
        q_ref[j] = acc * dinv


_tc_b = pl.pallas_call(
    _tc_b_body,
    out_shape=jax.ShapeDtypeStruct((D, PR, 128), jnp.float32),
)


def _tc_c_body(q0, q1, q2, q3, g_ref, dinv_ref, b2_ref, o_ref, l_ref):
    dinv = dinv_ref[...]
    o = []
    for j, q in enumerate((q0, q1, q2, q3)):
        o.append((q[0] + q[1] + g_ref[j]) * dinv + b2_ref[0:1, j:j + 1])
    m = jnp.maximum(jnp.maximum(o[0], o[1]), jnp.maximum(o[2], o[3]))
    z = [oj - m for oj in o]
    lse = jnp.log(jnp.exp(z[0]) + jnp.exp(z[1]) + jnp.exp(z[2]) + jnp.exp(z[3]))
    for j in range(D):
        o_ref[j] = o[j]
        l_ref[j] = z[j] - lse


_tc_c = pl.pallas_call(
    _tc_c_body,
    out_shape=[jax.ShapeDtypeStruct((D, PR, 128), jnp.float32)] * 2,
)


def kernel(x, edge_index, W1, b1, W2, b2):
    x_pad = jnp.pad(x, ((0, NPAD - N), (0, 0)))
    x3 = x_pad.reshape(PR, 128, D_IN)
    padi = (jnp.arange(EPAD - E, dtype=jnp.int32) % NTAIL) + N
    src_e = jnp.concatenate([edge_index[0], padi])
    dst_e = jnp.concatenate([edge_index[1], padi])
    ones = jnp.ones((EPT,), jnp.float32)
    zeros1 = jnp.zeros((NPAD,), jnp.float32)
    zeros8 = jnp.zeros((NPAD, DP), jnp.float32)

    degp = _deg_kernel(dst_e, ones, zeros1)
    degp3 = degp.reshape(NC, PR, 128)
    g1, dinv2 = _tc_a(degp3, x3, W1.T)
    p = _agg_kernel(g1, src_e, dst_e, zeros8)
    g2 = _tc_b(*p, g1, dinv2, W2, b1[None, :])
    q = _agg_kernel(g2, src_e, dst_e, zeros8)
    oT, lT = _tc_c(*q, g2, dinv2, b2[None, :])
    o = jnp.moveaxis(oT, 0, 2).reshape(NPAD, D)[:N]
    ls = jnp.moveaxis(lT, 0, 2).reshape(NPAD, D)[:N]
    return (o, ls)
